# baseline (speedup 1.0000x reference)
import functools

import jax
import jax.numpy as jnp
from jax import lax
from jax.experimental import pallas as pl
from jax.experimental.pallas import tpu as pltpu
from jax.experimental.pallas import tpu_sc as plsc

B = 16384
F = 32


def _make_gmf():
    info = plsc.get_sparse_core_info()
    NC, NS, L = info.num_cores, info.num_subcores, info.num_lanes
    NW = NC * NS
    BPW = B // NW

    mesh = plsc.VectorSubcoreMesh(core_axis_name="c", subcore_axis_name="s")

    @functools.partial(
        pl.kernel,
        mesh=mesh,
        compiler_params=pltpu.CompilerParams(
            use_tc_tiling_on_sc=False, needs_layout_passes=False
        ),
        out_type=jax.ShapeDtypeStruct((B,), jnp.float32),
        scratch_types=[
            pltpu.VMEM((BPW,), jnp.int32),
            pltpu.VMEM((BPW,), jnp.int32),
            pltpu.VMEM((BPW, F), jnp.float32),
            pltpu.VMEM((BPW, F), jnp.float32),
            pltpu.VMEM((F,), jnp.float32),
            pltpu.VMEM((16,), jnp.float32),
            pltpu.VMEM((BPW,), jnp.float32),
            pltpu.SemaphoreType.DMA,
            pltpu.SemaphoreType.DMA,
        ],
    )
    def gmf(user_hbm, item_hbm, ut_hbm, it_hbm, w_hbm, b_hbm, out_hbm,
            uidx_v, iidx_v, urows_v, irows_v, w_v, b_v, out_v, sem_u, sem_i):
        wid = lax.axis_index("s") * NC + lax.axis_index("c")
        base = wid * BPW
        pltpu.sync_copy(user_hbm.at[pl.ds(base, BPW)], uidx_v)
        pltpu.sync_copy(item_hbm.at[pl.ds(base, BPW)], iidx_v)
        pltpu.sync_copy(w_hbm, w_v)
        pltpu.sync_copy(b_hbm, b_v.at[pl.ds(0, 1)])
        cu = pltpu.async_copy(ut_hbm.at[uidx_v], urows_v, sem_u)
        ci = pltpu.async_copy(it_hbm.at[iidx_v], irows_v, sem_i)
        cu.wait()
        ci.wait()

        w_lo = w_v[pl.ds(0, 16)]
        w_hi = w_v[pl.ds(16, 16)]
        ws = [w_lo[f] for f in range(16)] + [w_hi[f] for f in range(16)]
        bias = b_v[pl.ds(0, 16)][0]
        lane = lax.iota(jnp.int32, 16)

        def group(g, carry):
            rows = g * 16 + lane
            acc = jnp.full((16,), bias, jnp.float32)
            for f in range(F):
                col = jnp.full((16,), f, jnp.int32)
                gu = plsc.load_gather(urows_v, [rows, col])
                gv = plsc.load_gather(irows_v, [rows, col])
                acc = acc + gu * gv * ws[f]
            out_v[pl.ds(g * 16, 16)] = 1.0 / (1.0 + jnp.exp(-acc))
            return carry

        lax.fori_loop(0, BPW // 16, group, 0)
        pltpu.sync_copy(out_v, out_hbm.at[pl.ds(base, BPW)])

    return gmf


_gmf = _make_gmf()


def kernel(user, item, user_table, item_table, W, b):
    return _gmf(user, item, user_table, item_table, W.reshape(F), b)


# speedup vs baseline: 3.3834x; 3.3834x over previous
import functools

import jax
import jax.numpy as jnp
from jax import lax
from jax.experimental import pallas as pl
from jax.experimental.pallas import tpu as pltpu
from jax.experimental.pallas import tpu_sc as plsc

B = 16384
F = 32
G = 16


def _make_gmf():
    info = plsc.get_sparse_core_info()
    NC, NS = info.num_cores, info.num_subcores
    NW = NC * NS
    BPW = B // NW
    NSG = BPW // G

    mesh = plsc.VectorSubcoreMesh(core_axis_name="c", subcore_axis_name="s")

    @functools.partial(
        pl.kernel,
        mesh=mesh,
        compiler_params=pltpu.CompilerParams(needs_layout_passes=False),
        out_type=jax.ShapeDtypeStruct((B,), jnp.float32),
        scratch_types=[
            pltpu.VMEM((BPW,), jnp.int32),
            pltpu.VMEM((BPW,), jnp.int32),
            pltpu.VMEM((8, F, 128), jnp.float32),
            pltpu.VMEM((8, F, 128), jnp.float32),
            pltpu.VMEM((G, F), jnp.float32),
            pltpu.VMEM((F,), jnp.float32),
            pltpu.VMEM((G,), jnp.float32),
            pltpu.VMEM((BPW,), jnp.float32),
            pltpu.SemaphoreType.DMA,
            pltpu.SemaphoreType.DMA,
        ],
    )
    def gmf(user_hbm, item_hbm, utt_hbm, itt_hbm, w_hbm, b_hbm, out_hbm,
            uidx_v, iidx_v, ublk, iblk, pstg, w_v, b_v, out_v, sem0, sem1):
        wid = lax.axis_index("s") * NC + lax.axis_index("c")
        base = wid * BPW
        pltpu.sync_copy(user_hbm.at[pl.ds(base, BPW)], uidx_v)
        pltpu.sync_copy(item_hbm.at[pl.ds(base, BPW)], iidx_v)
        pltpu.sync_copy(w_hbm, w_v)
        pltpu.sync_copy(b_hbm, b_v)

        w_lo = w_v[pl.ds(0, 16)]
        w_hi = w_v[pl.ds(16, 16)]
        bvec = b_v[...]
        lane = lax.iota(jnp.int32, 16)
        c_lo = lane
        c_hi = lane + 16
        sems = (sem0, sem1)

        def fetch(quad, rs_u, rs_i):
            half = quad % 2
            sem = sems[half]
            handles = []
            for e in range(4):
                slot = half * 4 + e
                ru = pl.multiple_of((rs_u[e] >> 7) << 7, 128)
                ri = pl.multiple_of((rs_i[e] >> 7) << 7, 128)
                handles.append(
                    pltpu.async_copy(
                        utt_hbm.at[:, pl.ds(ru, 128)], ublk.at[slot], sem
                    )
                )
                handles.append(
                    pltpu.async_copy(
                        itt_hbm.at[:, pl.ds(ri, 128)], iblk.at[slot], sem
                    )
                )
            return handles

        def extract(quad, rs_u, rs_i):
            half = quad % 2
            for e in range(4):
                j = quad * 4 + e
                slot = half * 4 + e
                slot_v = jnp.full((16,), slot, jnp.int32)
                rr_u = jnp.full((16,), rs_u[e] & 127, jnp.int32)
                rr_i = jnp.full((16,), rs_i[e] & 127, jnp.int32)
                gu_lo = plsc.load_gather(ublk, [slot_v, c_lo, rr_u])
                gu_hi = plsc.load_gather(ublk, [slot_v, c_hi, rr_u])
                gi_lo = plsc.load_gather(iblk, [slot_v, c_lo, rr_i])
                gi_hi = plsc.load_gather(iblk, [slot_v, c_hi, rr_i])
                pstg[j, pl.ds(0, 16)] = gu_lo * gi_lo * w_lo
                pstg[j, pl.ds(16, 16)] = gu_hi * gi_hi * w_hi

        def group(g, carry):
            uvec = uidx_v[pl.ds(g * G, G)]
            ivec = iidx_v[pl.ds(g * G, G)]
            rs_u = [uvec[j] for j in range(G)]
            rs_i = [ivec[j] for j in range(G)]
            handles = fetch(0, rs_u[0:4], rs_i[0:4])
            for quad in range(4):
                if quad < 3:
                    nxt = fetch(
                        quad + 1,
                        rs_u[4 * quad + 4:4 * quad + 8],
                        rs_i[4 * quad + 4:4 * quad + 8],
                    )
                else:
                    nxt = None
                for h in handles:
                    h.wait()
                extract(quad, rs_u[4 * quad:4 * quad + 4],
                        rs_i[4 * quad:4 * quad + 4])
                handles = nxt
            acc = bvec
            for c in range(F):
                acc = acc + plsc.load_gather(pstg, [lane, jnp.full((16,), c, jnp.int32)])
            out_v[pl.ds(g * G, G)] = 1.0 / (1.0 + jnp.exp(-acc))
            return carry

        lax.fori_loop(0, NSG, group, 0)
        pltpu.sync_copy(out_v, out_hbm.at[pl.ds(base, BPW)])

    return gmf


_gmf = _make_gmf()


def kernel(user, item, user_table, item_table, W, b):
    return _gmf(
        user,
        item,
        user_table.T,
        item_table.T,
        W.reshape(F),
        jnp.broadcast_to(b, (G,)),
    )


# speedup vs baseline: 3.4999x; 1.0344x over previous
import functools

import jax
import jax.numpy as jnp
from jax import lax
from jax.experimental import pallas as pl
from jax.experimental.pallas import tpu as pltpu
from jax.experimental.pallas import tpu_sc as plsc

B = 16384
F = 32
G = 16


def _make_gmf():
    info = plsc.get_sparse_core_info()
    NC, NS = info.num_cores, info.num_subcores
    NW = NC * NS
    BPW = B // NW
    NSG = BPW // G

    mesh = plsc.VectorSubcoreMesh(core_axis_name="c", subcore_axis_name="s")

    @functools.partial(
        pl.kernel,
        mesh=mesh,
        compiler_params=pltpu.CompilerParams(needs_layout_passes=False),
        out_type=jax.ShapeDtypeStruct((B,), jnp.float32),
        scratch_types=[
            pltpu.VMEM((BPW,), jnp.int32),
            pltpu.VMEM((BPW,), jnp.int32),
            pltpu.VMEM((8, F, 128), jnp.float32),
            pltpu.VMEM((8, F, 128), jnp.float32),
            pltpu.VMEM((G, F), jnp.float32),
            pltpu.VMEM((F,), jnp.float32),
            pltpu.VMEM((G,), jnp.float32),
            pltpu.VMEM((BPW,), jnp.float32),
            pltpu.SemaphoreType.DMA,
            pltpu.SemaphoreType.DMA,
        ],
    )
    def gmf(user_hbm, item_hbm, utt_hbm, itt_hbm, w_hbm, b_hbm, out_hbm,
            uidx_v, iidx_v, ublk, iblk, pstg, w_v, b_v, out_v, sem0, sem1):
        wid = lax.axis_index("s") * NC + lax.axis_index("c")
        base = wid * BPW
        pltpu.sync_copy(user_hbm.at[pl.ds(base, BPW)], uidx_v)
        pltpu.sync_copy(item_hbm.at[pl.ds(base, BPW)], iidx_v)
        pltpu.sync_copy(w_hbm, w_v)
        pltpu.sync_copy(b_hbm, b_v)

        w_lo = w_v[pl.ds(0, 16)]
        w_hi = w_v[pl.ds(16, 16)]
        bvec = b_v[...]
        lane = lax.iota(jnp.int32, 16)
        c_lo = lane
        c_hi = lane + 16
        sems = (sem0, sem1)

        def fetch(par, rs_u, rs_i):
            sem = sems[par]
            for e in range(4):
                slot = par * 4 + e
                ru = pl.multiple_of((rs_u[e] >> 7) << 7, 128)
                ri = pl.multiple_of((rs_i[e] >> 7) << 7, 128)
                pltpu.async_copy(
                    utt_hbm.at[:, pl.ds(ru, 128)], ublk.at[slot], sem
                )
                pltpu.async_copy(
                    itt_hbm.at[:, pl.ds(ri, 128)], iblk.at[slot], sem
                )

        def drain(par):
            sem = sems[par]
            for e in range(4):
                slot = par * 4 + e
                pltpu.make_async_copy(
                    utt_hbm.at[:, pl.ds(0, 128)], ublk.at[slot], sem
                ).wait()
                pltpu.make_async_copy(
                    itt_hbm.at[:, pl.ds(0, 128)], iblk.at[slot], sem
                ).wait()

        def extract(quad, rs_u, rs_i):
            par = quad % 2
            for e in range(4):
                j = quad * 4 + e
                slot = par * 4 + e
                slot_v = jnp.full((16,), slot, jnp.int32)
                rr_u = jnp.full((16,), rs_u[e] & 127, jnp.int32)
                rr_i = jnp.full((16,), rs_i[e] & 127, jnp.int32)
                gu_lo = plsc.load_gather(ublk, [slot_v, c_lo, rr_u])
                gu_hi = plsc.load_gather(ublk, [slot_v, c_hi, rr_u])
                gi_lo = plsc.load_gather(iblk, [slot_v, c_lo, rr_i])
                gi_hi = plsc.load_gather(iblk, [slot_v, c_hi, rr_i])
                pstg[j, pl.ds(0, 16)] = gu_lo * gi_lo * w_lo
                pstg[j, pl.ds(16, 16)] = gu_hi * gi_hi * w_hi

        def load_vecs(g):
            uvec = uidx_v[pl.ds(g * G, G)]
            ivec = iidx_v[pl.ds(g * G, G)]
            return uvec, ivec

        uvec0, ivec0 = load_vecs(0)
        rs_u0 = [uvec0[j] for j in range(G)]
        rs_i0 = [ivec0[j] for j in range(G)]
        fetch(0, rs_u0[0:4], rs_i0[0:4])
        fetch(1, rs_u0[4:8], rs_i0[4:8])

        def group(g, carry):
            uvec, ivec = carry
            gn = jnp.minimum(g + 1, NSG - 1)
            uvn, ivn = load_vecs(gn)
            last = g + 1 >= NSG
            rs_u = [uvec[j] for j in range(G)]
            rs_i = [ivec[j] for j in range(G)]
            rsn_u = [jnp.where(last, 0, uvn[j]) for j in range(8)]
            rsn_i = [jnp.where(last, 0, ivn[j]) for j in range(8)]
            for k in range(4):
                par = k % 2
                drain(par)
                extract(k, rs_u[4 * k:4 * k + 4], rs_i[4 * k:4 * k + 4])
                if k < 2:
                    fetch(par, rs_u[4 * k + 8:4 * k + 12],
                          rs_i[4 * k + 8:4 * k + 12])
                else:
                    fetch(par, rsn_u[4 * (k - 2):4 * (k - 2) + 4],
                          rsn_i[4 * (k - 2):4 * (k - 2) + 4])
            acc = bvec
            for c in range(F):
                acc = acc + plsc.load_gather(
                    pstg, [lane, jnp.full((16,), c, jnp.int32)]
                )
            out_v[pl.ds(g * G, G)] = 1.0 / (1.0 + jnp.exp(-acc))
            return (uvn, ivn)

        lax.fori_loop(0, NSG, group, (uvec0, ivec0))
        drain(0)
        drain(1)
        pltpu.sync_copy(out_v, out_hbm.at[pl.ds(base, BPW)])

    return gmf


_gmf = _make_gmf()


def kernel(user, item, user_table, item_table, W, b):
    return _gmf(
        user,
        item,
        user_table.T,
        item_table.T,
        W.reshape(F),
        jnp.broadcast_to(b, (G,)),
    )
